# speedup vs baseline: 251.8359x; 251.8359x over previous
import functools

import jax
import jax.numpy as jnp
import numpy as np
from jax import lax
from jax.experimental import pallas as pl
from jax.experimental.pallas import tpu as pltpu
from jax.experimental.pallas import tpu_sc as plsc

H = 8
DK = 32
D = H * DK
VOCAB = 32000
DFF = 1024
N = 512
E = 65536

NC = 2
NS = 16
NW = NC * NS
EPW = E // NW
NBINS = N * N
TBINS = 3 * NBINS
SLICE = TBINS // NS
ROWS_PW = N // NW


def _sc_body(ee_src, ee_dst, dd_src, dd_dst, ed_src, ed_dst,
             src_tok, tgt_tok, pos_tab,
             src_tokens, src_pos, tgt_tokens, tgt_pos,
             cnt_out, x_enc_out, x_dec_out,
             srcbuf, dstbuf, idx_v, ones_v, zbuf,
             tokidx, posidx, trows, prows, cnt_sh, sem):
    c = lax.axis_index("c")
    s = lax.axis_index("s")
    wid = c * NS + s

    for i in range(8):
        ones_v[pl.ds(i * 16, 16)] = jnp.ones((16,), jnp.float32)

    def _z(i, _):
        zbuf[pl.ds(i * 16, 16)] = jnp.zeros((16,), jnp.float32)
        return 0
    lax.fori_loop(0, 128, _z, 0)

    for i in range(SLICE // 2048):
        pltpu.sync_copy(zbuf, cnt_sh.at[pl.ds(s * SLICE + i * 2048, 2048)])

    def _embed(tok_tab, tok_ids, pos_ids, out_ref):
        base = wid * ROWS_PW
        pltpu.sync_copy(tok_ids.at[pl.ds(base, ROWS_PW)], tokidx)
        pltpu.sync_copy(pos_ids.at[pl.ds(base, ROWS_PW)], posidx)
        pltpu.async_copy(tok_tab.at[tokidx], trows, sem).wait()
        pltpu.async_copy(pos_tab.at[posidx], prows, sem).wait()

        def _row(i, _):
            def _col(j, _):
                trows[i, pl.ds(j * 16, 16)] = (
                    trows[i, pl.ds(j * 16, 16)] + prows[i, pl.ds(j * 16, 16)])
                return 0
            lax.fori_loop(0, D // 16, _col, 0)
            return 0
        lax.fori_loop(0, ROWS_PW, _row, 0)
        pltpu.sync_copy(trows, out_ref.at[pl.ds(base, ROWS_PW)])

    _embed(src_tok, src_tokens, src_pos, x_enc_out)
    _embed(tgt_tok, tgt_tokens, tgt_pos, x_dec_out)

    plsc.subcore_barrier()

    ebase = wid * EPW
    for t, (esrc, edst) in enumerate(((ee_src, ee_dst),
                                      (dd_src, dd_dst),
                                      (ed_src, ed_dst))):
        pltpu.sync_copy(esrc.at[pl.ds(ebase, EPW)], srcbuf)
        pltpu.sync_copy(edst.at[pl.ds(ebase, EPW)], dstbuf)
        for j in range(16):
            for k in range(8):
                off = (j * 8 + k) * 16
                idx_v[j, pl.ds(k * 16, 16)] = (
                    dstbuf[pl.ds(off, 16)] * N
                    + srcbuf[pl.ds(off, 16)]
                    + t * NBINS)
        descs = [pltpu.async_copy(ones_v, cnt_sh.at[idx_v.at[j]], sem,
                                  add=True)
                 for j in range(16)]
        for d in descs:
            d.wait()

    plsc.subcore_barrier()

    pltpu.sync_copy(cnt_sh.at[pl.ds(s * SLICE, SLICE)],
                    cnt_out.at[pl.ds(c * TBINS + s * SLICE, SLICE)])


def _sc_prep(ee_src, ee_dst, dd_src, dd_dst, ed_src, ed_dst,
             src_tok, tgt_tok, pos_tab,
             src_tokens, src_pos, tgt_tokens, tgt_pos):
    mesh = plsc.VectorSubcoreMesh(core_axis_name="c", subcore_axis_name="s",
                                  num_cores=NC, num_subcores=NS)
    f = pl.kernel(
        _sc_body,
        out_type=(
            jax.ShapeDtypeStruct((NC * TBINS,), jnp.float32),
            jax.ShapeDtypeStruct((N, D), jnp.float32),
            jax.ShapeDtypeStruct((N, D), jnp.float32),
        ),
        mesh=mesh,
        scratch_types=(
            pltpu.VMEM((EPW,), jnp.int32),
            pltpu.VMEM((EPW,), jnp.int32),
            pltpu.VMEM((16, 128), jnp.int32),
            pltpu.VMEM((128,), jnp.float32),
            pltpu.VMEM((2048,), jnp.float32),
            pltpu.VMEM((ROWS_PW,), jnp.int32),
            pltpu.VMEM((ROWS_PW,), jnp.int32),
            pltpu.VMEM((ROWS_PW, D), jnp.float32),
            pltpu.VMEM((ROWS_PW, D), jnp.float32),
            pltpu.VMEM_SHARED((TBINS,), jnp.float32),
            pltpu.SemaphoreType.DMA,
        ),
    )
    return f(ee_src, ee_dst, dd_src, dd_dst, ed_src, ed_dst,
             src_tok, tgt_tok, pos_tab,
             src_tokens, src_pos, tgt_tokens, tgt_pos)


def _layernorm(x, g, b):
    m = jnp.mean(x, axis=-1, keepdims=True)
    v = jnp.mean((x - m) ** 2, axis=-1, keepdims=True)
    return (x - m) / jnp.sqrt(v + 1e-5) * g + b


def _attn(xq, xkv, C, Wq, Wk, Wv, Wo):
    q = jnp.dot(xq, Wq, preferred_element_type=jnp.float32)
    k = jnp.dot(xkv, Wk, preferred_element_type=jnp.float32)
    v = jnp.dot(xkv, Wv, preferred_element_type=jnp.float32)
    outs = []
    inv = np.float32(1.0 / np.sqrt(DK))
    for h in range(H):
        qh = q[:, h * DK:(h + 1) * DK]
        kh = k[:, h * DK:(h + 1) * DK]
        vh = v[:, h * DK:(h + 1) * DK]
        S = lax.dot_general(qh, kh, (((1,), (1,)), ((), ())),
                            preferred_element_type=jnp.float32) * inv
        W = jnp.exp(jnp.clip(S, -10.0, 10.0)) * C
        wv = jnp.dot(W, vh, preferred_element_type=jnp.float32)
        z = jnp.sum(W, axis=1, keepdims=True)
        outs.append(wv / (z + 1e-9))
    o = jnp.concatenate(outs, axis=1)
    return jnp.dot(o, Wo, preferred_element_type=jnp.float32)


def _ffn(x, W1, b1, W2, b2):
    h = jax.nn.relu(jnp.dot(x, W1, preferred_element_type=jnp.float32) + b1)
    return jnp.dot(h, W2, preferred_element_type=jnp.float32) + b2


def _body_kernel(enc_tree, dec_tree, n_enc, *refs):
    x_enc_ref, x_dec_ref, cnt_ref = refs[0], refs[1], refs[2]
    enc_refs = refs[3:3 + n_enc]
    dec_refs = refs[3 + n_enc:-1]
    out_ref = refs[-1]
    enc_params = jax.tree.unflatten(enc_tree, enc_refs)
    dec_params = jax.tree.unflatten(dec_tree, dec_refs)

    Cee = cnt_ref[0, 0] + cnt_ref[1, 0]
    Cdd = cnt_ref[0, 1] + cnt_ref[1, 1]
    Ced = cnt_ref[0, 2] + cnt_ref[1, 2]

    x = x_enc_ref[...]
    for p in enc_params:
        x = _layernorm(
            x + _attn(x, x, Cee, p['Wq'][...], p['Wk'][...], p['Wv'][...],
                      p['Wo'][...]),
            p['ln1_g'][...], p['ln1_b'][...])
        x = _layernorm(x + _ffn(x, p['W1'][...], p['b1'][...],
                                p['W2'][...], p['b2'][...]),
                       p['ln2_g'][...], p['ln2_b'][...])
    x_enc = x

    x = x_dec_ref[...]
    for p in dec_params:
        x = _layernorm(
            x + _attn(x, x, Cdd, p['Wq'][...], p['Wk'][...], p['Wv'][...],
                      p['Wo'][...]),
            p['ln1_g'][...], p['ln1_b'][...])
        x = _layernorm(
            x + _attn(x, x_enc, Ced, p['Wq2'][...], p['Wk2'][...],
                      p['Wv2'][...], p['Wo2'][...]),
            p['ln2_g'][...], p['ln2_b'][...])
        x = _layernorm(x + _ffn(x, p['W1'][...], p['b1'][...],
                                p['W2'][...], p['b2'][...]),
                       p['ln3_g'][...], p['ln3_b'][...])
    out_ref[...] = x


def _body(x_enc0, x_dec0, cnt, enc_params, dec_params):
    enc_leaves, enc_tree = jax.tree.flatten(enc_params)
    dec_leaves, dec_tree = jax.tree.flatten(dec_params)
    return pl.pallas_call(
        functools.partial(_body_kernel, enc_tree, dec_tree, len(enc_leaves)),
        out_shape=jax.ShapeDtypeStruct((N, D), jnp.float32),
        compiler_params=pltpu.CompilerParams(
            vmem_limit_bytes=100 * 1024 * 1024),
    )(x_enc0, x_dec0, cnt, *enc_leaves, *dec_leaves)


BV = 1280
KV = VOCAB // BV


def _logz_kernel(x_ref, wg_ref, bg_ref, out_ref, m_sc, s_sc):
    j = pl.program_id(0)
    l = jnp.dot(x_ref[...], wg_ref[...],
                preferred_element_type=jnp.float32) + bg_ref[...]
    bm = jnp.max(l, axis=1, keepdims=True)

    @pl.when(j == 0)
    def _():
        m_sc[...] = bm
        s_sc[...] = jnp.sum(jnp.exp(l - bm), axis=1, keepdims=True)

    @pl.when(j > 0)
    def _():
        m_old = m_sc[...]
        m_new = jnp.maximum(m_old, bm)
        s_sc[...] = (s_sc[...] * jnp.exp(m_old - m_new)
                     + jnp.sum(jnp.exp(l - m_new), axis=1, keepdims=True))
        m_sc[...] = m_new

    @pl.when(j == KV - 1)
    def _():
        out_ref[...] = m_sc[...] + jnp.log(s_sc[...])


def _gen_out_kernel(x_ref, wg_ref, bg_ref, lz_ref, out_ref):
    l = jnp.dot(x_ref[...], wg_ref[...],
                preferred_element_type=jnp.float32) + bg_ref[...]
    out_ref[...] = l - lz_ref[...]


def _generator(x_dec, Wg, bg):
    xb = x_dec.astype(jnp.bfloat16)
    wgb = Wg.astype(jnp.bfloat16)
    bg2 = bg.reshape(1, VOCAB)
    logz = pl.pallas_call(
        _logz_kernel,
        grid=(KV,),
        in_specs=[
            pl.BlockSpec((N, D), lambda j: (0, 0)),
            pl.BlockSpec((D, BV), lambda j: (0, j)),
            pl.BlockSpec((1, BV), lambda j: (0, j)),
        ],
        out_specs=pl.BlockSpec((N, 1), lambda j: (0, 0)),
        out_shape=jax.ShapeDtypeStruct((N, 1), jnp.float32),
        scratch_shapes=[pltpu.VMEM((N, 1), jnp.float32),
                        pltpu.VMEM((N, 1), jnp.float32)],
    )(xb, wgb, bg2)
    return pl.pallas_call(
        _gen_out_kernel,
        grid=(KV,),
        in_specs=[
            pl.BlockSpec((N, D), lambda j: (0, 0)),
            pl.BlockSpec((D, BV), lambda j: (0, j)),
            pl.BlockSpec((1, BV), lambda j: (0, j)),
            pl.BlockSpec((N, 1), lambda j: (0, 0)),
        ],
        out_specs=pl.BlockSpec((N, BV), lambda j: (0, j)),
        out_shape=jax.ShapeDtypeStruct((N, VOCAB), jnp.float32),
    )(xb, wgb, bg2, logz)


def kernel(params, src_tokens, src_pos, tgt_tokens, tgt_pos,
           ee_src, ee_dst, dd_src, dd_dst, ed_src, ed_dst):
    cnt_flat, x_enc0, x_dec0 = _sc_prep(
        ee_src, ee_dst, dd_src, dd_dst, ed_src, ed_dst,
        params['src_tok'], params['tgt_tok'], params['pos'],
        src_tokens, src_pos, tgt_tokens, tgt_pos)
    cnt = cnt_flat.reshape(NC, 3, N, N)
    x_dec = _body(x_enc0, x_dec0, cnt, params['enc'], params['dec'])
    return _generator(x_dec, params['Wg'], params['bg'])


# speedup vs baseline: 251.9022x; 1.0003x over previous
import functools

import jax
import jax.numpy as jnp
import numpy as np
from jax import lax
from jax.experimental import pallas as pl
from jax.experimental.pallas import tpu as pltpu
from jax.experimental.pallas import tpu_sc as plsc

H = 8
DK = 32
D = H * DK
VOCAB = 32000
DFF = 1024
N = 512
E = 65536

NC = 2
NS = 16
NW = NC * NS
EPW = E // NW
NBINS = N * N
TBINS = 3 * NBINS
SLICE = TBINS // NS
ROWS_PW = N // NW


def _sc_body(ee_src, ee_dst, dd_src, dd_dst, ed_src, ed_dst,
             src_tok, tgt_tok, pos_tab,
             src_tokens, src_pos, tgt_tokens, tgt_pos,
             cnt_out, x_enc_out, x_dec_out,
             srcbuf, dstbuf, idx_v, ones_v, zbuf,
             tokidx, posidx, trows, prows, cnt_sh, sem):
    c = lax.axis_index("c")
    s = lax.axis_index("s")
    wid = c * NS + s

    for i in range(8):
        ones_v[pl.ds(i * 16, 16)] = jnp.ones((16,), jnp.float32)

    def _z(i, _):
        zbuf[pl.ds(i * 16, 16)] = jnp.zeros((16,), jnp.float32)
        return 0
    lax.fori_loop(0, 128, _z, 0)

    for i in range(SLICE // 2048):
        pltpu.sync_copy(zbuf, cnt_sh.at[pl.ds(s * SLICE + i * 2048, 2048)])

    def _embed(tok_tab, tok_ids, pos_ids, out_ref):
        base = wid * ROWS_PW
        pltpu.sync_copy(tok_ids.at[pl.ds(base, ROWS_PW)], tokidx)
        pltpu.sync_copy(pos_ids.at[pl.ds(base, ROWS_PW)], posidx)
        pltpu.async_copy(tok_tab.at[tokidx], trows, sem).wait()
        pltpu.async_copy(pos_tab.at[posidx], prows, sem).wait()

        def _row(i, _):
            def _col(j, _):
                trows[i, pl.ds(j * 16, 16)] = (
                    trows[i, pl.ds(j * 16, 16)] + prows[i, pl.ds(j * 16, 16)])
                return 0
            lax.fori_loop(0, D // 16, _col, 0)
            return 0
        lax.fori_loop(0, ROWS_PW, _row, 0)
        pltpu.sync_copy(trows, out_ref.at[pl.ds(base, ROWS_PW)])

    _embed(src_tok, src_tokens, src_pos, x_enc_out)
    _embed(tgt_tok, tgt_tokens, tgt_pos, x_dec_out)

    plsc.subcore_barrier()

    ebase = wid * EPW
    for t, (esrc, edst) in enumerate(((ee_src, ee_dst),
                                      (dd_src, dd_dst),
                                      (ed_src, ed_dst))):
        pltpu.sync_copy(esrc.at[pl.ds(ebase, EPW)], srcbuf)
        pltpu.sync_copy(edst.at[pl.ds(ebase, EPW)], dstbuf)
        for j in range(16):
            for k in range(8):
                off = (j * 8 + k) * 16
                idx_v[j, pl.ds(k * 16, 16)] = (
                    dstbuf[pl.ds(off, 16)] * N
                    + srcbuf[pl.ds(off, 16)]
                    + t * NBINS)
        descs = [pltpu.async_copy(ones_v, cnt_sh.at[idx_v.at[j]], sem,
                                  add=True)
                 for j in range(16)]
        for d in descs:
            d.wait()

    plsc.subcore_barrier()

    pltpu.sync_copy(cnt_sh.at[pl.ds(s * SLICE, SLICE)],
                    cnt_out.at[pl.ds(c * TBINS + s * SLICE, SLICE)])


def _sc_prep(ee_src, ee_dst, dd_src, dd_dst, ed_src, ed_dst,
             src_tok, tgt_tok, pos_tab,
             src_tokens, src_pos, tgt_tokens, tgt_pos):
    mesh = plsc.VectorSubcoreMesh(core_axis_name="c", subcore_axis_name="s",
                                  num_cores=NC, num_subcores=NS)
    f = pl.kernel(
        _sc_body,
        out_type=(
            jax.ShapeDtypeStruct((NC * TBINS,), jnp.float32),
            jax.ShapeDtypeStruct((N, D), jnp.float32),
            jax.ShapeDtypeStruct((N, D), jnp.float32),
        ),
        mesh=mesh,
        scratch_types=(
            pltpu.VMEM((EPW,), jnp.int32),
            pltpu.VMEM((EPW,), jnp.int32),
            pltpu.VMEM((16, 128), jnp.int32),
            pltpu.VMEM((128,), jnp.float32),
            pltpu.VMEM((2048,), jnp.float32),
            pltpu.VMEM((ROWS_PW,), jnp.int32),
            pltpu.VMEM((ROWS_PW,), jnp.int32),
            pltpu.VMEM((ROWS_PW, D), jnp.float32),
            pltpu.VMEM((ROWS_PW, D), jnp.float32),
            pltpu.VMEM_SHARED((TBINS,), jnp.float32),
            pltpu.SemaphoreType.DMA,
        ),
    )
    return f(ee_src, ee_dst, dd_src, dd_dst, ed_src, ed_dst,
             src_tok, tgt_tok, pos_tab,
             src_tokens, src_pos, tgt_tokens, tgt_pos)


def _mm(a, b):
    return jnp.dot(a.astype(jnp.bfloat16), b.astype(jnp.bfloat16),
                   preferred_element_type=jnp.float32)


def _layernorm(x, g, b):
    m = jnp.mean(x, axis=-1, keepdims=True)
    v = jnp.mean((x - m) ** 2, axis=-1, keepdims=True)
    return (x - m) / jnp.sqrt(v + 1e-5) * g + b


def _attn(xq, xkv, C, Wq, Wk, Wv, Wo):
    q = _mm(xq, Wq)
    k = _mm(xkv, Wk)
    v = _mm(xkv, Wv)
    outs = []
    inv = np.float32(1.0 / np.sqrt(DK))
    for h in range(H):
        qh = q[:, h * DK:(h + 1) * DK]
        kh = k[:, h * DK:(h + 1) * DK]
        vh = v[:, h * DK:(h + 1) * DK]
        S = lax.dot_general(qh.astype(jnp.bfloat16), kh.astype(jnp.bfloat16),
                            (((1,), (1,)), ((), ())),
                            preferred_element_type=jnp.float32) * inv
        W = jnp.exp(jnp.clip(S, -10.0, 10.0)) * C
        wv = _mm(W, vh)
        z = jnp.sum(W, axis=1, keepdims=True)
        outs.append(wv / (z + 1e-9))
    o = jnp.concatenate(outs, axis=1)
    return _mm(o, Wo)


def _ffn(x, W1, b1, W2, b2):
    h = jax.nn.relu(_mm(x, W1) + b1)
    return _mm(h, W2) + b2


def _body_kernel(enc_tree, dec_tree, n_enc, *refs):
    x_enc_ref, x_dec_ref, cnt_ref = refs[0], refs[1], refs[2]
    enc_refs = refs[3:3 + n_enc]
    dec_refs = refs[3 + n_enc:-1]
    out_ref = refs[-1]
    enc_params = jax.tree.unflatten(enc_tree, enc_refs)
    dec_params = jax.tree.unflatten(dec_tree, dec_refs)

    Cee = cnt_ref[0, 0] + cnt_ref[1, 0]
    Cdd = cnt_ref[0, 1] + cnt_ref[1, 1]
    Ced = cnt_ref[0, 2] + cnt_ref[1, 2]

    x = x_enc_ref[...]
    for p in enc_params:
        x = _layernorm(
            x + _attn(x, x, Cee, p['Wq'][...], p['Wk'][...], p['Wv'][...],
                      p['Wo'][...]),
            p['ln1_g'][...], p['ln1_b'][...])
        x = _layernorm(x + _ffn(x, p['W1'][...], p['b1'][...],
                                p['W2'][...], p['b2'][...]),
                       p['ln2_g'][...], p['ln2_b'][...])
    x_enc = x

    x = x_dec_ref[...]
    for p in dec_params:
        x = _layernorm(
            x + _attn(x, x, Cdd, p['Wq'][...], p['Wk'][...], p['Wv'][...],
                      p['Wo'][...]),
            p['ln1_g'][...], p['ln1_b'][...])
        x = _layernorm(
            x + _attn(x, x_enc, Ced, p['Wq2'][...], p['Wk2'][...],
                      p['Wv2'][...], p['Wo2'][...]),
            p['ln2_g'][...], p['ln2_b'][...])
        x = _layernorm(x + _ffn(x, p['W1'][...], p['b1'][...],
                                p['W2'][...], p['b2'][...]),
                       p['ln3_g'][...], p['ln3_b'][...])
    out_ref[...] = x


def _body(x_enc0, x_dec0, cnt, enc_params, dec_params):
    enc_leaves, enc_tree = jax.tree.flatten(enc_params)
    dec_leaves, dec_tree = jax.tree.flatten(dec_params)
    return pl.pallas_call(
        functools.partial(_body_kernel, enc_tree, dec_tree, len(enc_leaves)),
        out_shape=jax.ShapeDtypeStruct((N, D), jnp.float32),
        compiler_params=pltpu.CompilerParams(
            vmem_limit_bytes=100 * 1024 * 1024),
    )(x_enc0, x_dec0, cnt, *enc_leaves, *dec_leaves)


BV = 1280
KV = VOCAB // BV


def _logz_kernel(x_ref, wg_ref, bg_ref, out_ref, m_sc, s_sc):
    j = pl.program_id(0)
    l = jnp.dot(x_ref[...], wg_ref[...],
                preferred_element_type=jnp.float32) + bg_ref[...]
    bm = jnp.max(l, axis=1, keepdims=True)

    @pl.when(j == 0)
    def _():
        m_sc[...] = bm
        s_sc[...] = jnp.sum(jnp.exp(l - bm), axis=1, keepdims=True)

    @pl.when(j > 0)
    def _():
        m_old = m_sc[...]
        m_new = jnp.maximum(m_old, bm)
        s_sc[...] = (s_sc[...] * jnp.exp(m_old - m_new)
                     + jnp.sum(jnp.exp(l - m_new), axis=1, keepdims=True))
        m_sc[...] = m_new

    @pl.when(j == KV - 1)
    def _():
        out_ref[...] = m_sc[...] + jnp.log(s_sc[...])


def _gen_out_kernel(x_ref, wg_ref, bg_ref, lz_ref, out_ref):
    l = jnp.dot(x_ref[...], wg_ref[...],
                preferred_element_type=jnp.float32) + bg_ref[...]
    out_ref[...] = l - lz_ref[...]


def _generator(x_dec, Wg, bg):
    xb = x_dec.astype(jnp.bfloat16)
    wgb = Wg.astype(jnp.bfloat16)
    bg2 = bg.reshape(1, VOCAB)
    logz = pl.pallas_call(
        _logz_kernel,
        grid=(KV,),
        in_specs=[
            pl.BlockSpec((N, D), lambda j: (0, 0)),
            pl.BlockSpec((D, BV), lambda j: (0, j)),
            pl.BlockSpec((1, BV), lambda j: (0, j)),
        ],
        out_specs=pl.BlockSpec((N, 1), lambda j: (0, 0)),
        out_shape=jax.ShapeDtypeStruct((N, 1), jnp.float32),
        scratch_shapes=[pltpu.VMEM((N, 1), jnp.float32),
                        pltpu.VMEM((N, 1), jnp.float32)],
    )(xb, wgb, bg2)
    return pl.pallas_call(
        _gen_out_kernel,
        grid=(KV,),
        in_specs=[
            pl.BlockSpec((N, D), lambda j: (0, 0)),
            pl.BlockSpec((D, BV), lambda j: (0, j)),
            pl.BlockSpec((1, BV), lambda j: (0, j)),
            pl.BlockSpec((N, 1), lambda j: (0, 0)),
        ],
        out_specs=pl.BlockSpec((N, BV), lambda j: (0, j)),
        out_shape=jax.ShapeDtypeStruct((N, VOCAB), jnp.float32),
    )(xb, wgb, bg2, logz)


def kernel(params, src_tokens, src_pos, tgt_tokens, tgt_pos,
           ee_src, ee_dst, dd_src, dd_dst, ed_src, ed_dst):
    cnt_flat, x_enc0, x_dec0 = _sc_prep(
        ee_src, ee_dst, dd_src, dd_dst, ed_src, ed_dst,
        params['src_tok'], params['tgt_tok'], params['pos'],
        src_tokens, src_pos, tgt_tokens, tgt_pos)
    cnt = cnt_flat.reshape(NC, 3, N, N)
    x_dec = _body(x_enc0, x_dec0, cnt, params['enc'], params['dec'])
    return _generator(x_dec, params['Wg'], params['bg'])


# speedup vs baseline: 273.9141x; 1.0874x over previous
import functools

import jax
import jax.numpy as jnp
import numpy as np
from jax import lax
from jax.experimental import pallas as pl
from jax.experimental.pallas import tpu as pltpu
from jax.experimental.pallas import tpu_sc as plsc

H = 8
DK = 32
D = H * DK
VOCAB = 32000
DFF = 1024
N = 512
E = 65536

NC = 2
NS = 16
NW = NC * NS
EPW = E // NW
NBINS = N * N
TBINS = 3 * NBINS
SLICE = TBINS // NS
ROWS_PW = N // NW


def _sc_body(ee_src, ee_dst, dd_src, dd_dst, ed_src, ed_dst,
             src_tok, tgt_tok, pos_tab,
             src_tokens, src_pos, tgt_tokens, tgt_pos,
             ones_in, zeros_in,
             cnt_out, x_enc_out, x_dec_out,
             srcbuf, dstbuf, idx_v, ones_v,
             tokidx, posidx, trows, prows, cnt_sh, sem):
    c = lax.axis_index("c")
    s = lax.axis_index("s")
    wid = c * NS + s
    zrows = SLICE // N

    pltpu.sync_copy(ones_in, ones_v)
    pltpu.sync_copy(zeros_in, cnt_sh.at[pl.ds(s * SLICE, SLICE)])

    def _embed(tok_tab, tok_ids, pos_ids, out_ref):
        base = wid * ROWS_PW
        pltpu.sync_copy(tok_ids.at[pl.ds(base, ROWS_PW)], tokidx)
        pltpu.sync_copy(pos_ids.at[pl.ds(base, ROWS_PW)], posidx)
        pltpu.async_copy(tok_tab.at[tokidx], trows, sem).wait()
        pltpu.async_copy(pos_tab.at[posidx], prows, sem).wait()

        def _row(i, _):
            def _col(j, _):
                trows[i, pl.ds(j * 16, 16)] = (
                    trows[i, pl.ds(j * 16, 16)] + prows[i, pl.ds(j * 16, 16)])
                return 0
            lax.fori_loop(0, D // 16, _col, 0)
            return 0
        lax.fori_loop(0, ROWS_PW, _row, 0)
        pltpu.sync_copy(trows, out_ref.at[pl.ds(base, ROWS_PW)])

    _embed(src_tok, src_tokens, src_pos, x_enc_out)
    _embed(tgt_tok, tgt_tokens, tgt_pos, x_dec_out)

    plsc.subcore_barrier()

    ebase = wid * EPW
    for t, (esrc, edst) in enumerate(((ee_src, ee_dst),
                                      (dd_src, dd_dst),
                                      (ed_src, ed_dst))):
        pltpu.sync_copy(esrc.at[pl.ds(ebase, EPW)], srcbuf)
        pltpu.sync_copy(edst.at[pl.ds(ebase, EPW)], dstbuf)
        for j in range(16):
            for k in range(8):
                off = (j * 8 + k) * 16
                idx_v[j, pl.ds(k * 16, 16)] = (
                    dstbuf[pl.ds(off, 16)] * N
                    + srcbuf[pl.ds(off, 16)]
                    + t * NBINS)
        descs = [pltpu.async_copy(ones_v, cnt_sh.at[idx_v.at[j]], sem,
                                  add=True)
                 for j in range(16)]
        for d in descs:
            d.wait()

    plsc.subcore_barrier()

    odescs = [pltpu.async_copy(
        cnt_sh.at[pl.ds((s * zrows + r) * N, N)],
        cnt_out.at[c, s * zrows + r], sem)
        for r in range(zrows)]
    for dsc in odescs:
        dsc.wait()


def _sc_prep(ee_src, ee_dst, dd_src, dd_dst, ed_src, ed_dst,
             src_tok, tgt_tok, pos_tab,
             src_tokens, src_pos, tgt_tokens, tgt_pos):
    mesh = plsc.VectorSubcoreMesh(core_axis_name="c", subcore_axis_name="s",
                                  num_cores=NC, num_subcores=NS)
    f = pl.kernel(
        _sc_body,
        out_type=(
            jax.ShapeDtypeStruct((NC, 3 * N, N), jnp.float32),
            jax.ShapeDtypeStruct((N, D), jnp.float32),
            jax.ShapeDtypeStruct((N, D), jnp.float32),
        ),
        mesh=mesh,
        scratch_types=(
            pltpu.VMEM((EPW,), jnp.int32),
            pltpu.VMEM((EPW,), jnp.int32),
            pltpu.VMEM((16, 128), jnp.int32),
            pltpu.VMEM((128,), jnp.float32),
            pltpu.VMEM((ROWS_PW,), jnp.int32),
            pltpu.VMEM((ROWS_PW,), jnp.int32),
            pltpu.VMEM((ROWS_PW, D), jnp.float32),
            pltpu.VMEM((ROWS_PW, D), jnp.float32),
            pltpu.VMEM_SHARED((TBINS,), jnp.float32),
            pltpu.SemaphoreType.DMA,
        ),
    )
    ones_in = jnp.ones((128,), jnp.float32)
    zeros_in = jnp.zeros((SLICE,), jnp.float32)
    return f(ee_src, ee_dst, dd_src, dd_dst, ed_src, ed_dst,
             src_tok, tgt_tok, pos_tab,
             src_tokens, src_pos, tgt_tokens, tgt_pos,
             ones_in, zeros_in)


def _mm(a, b):
    return jnp.dot(a.astype(jnp.bfloat16), b.astype(jnp.bfloat16),
                   preferred_element_type=jnp.float32)


def _layernorm(x, g, b):
    m = jnp.mean(x, axis=-1, keepdims=True)
    v = jnp.mean((x - m) ** 2, axis=-1, keepdims=True)
    return (x - m) / jnp.sqrt(v + 1e-5) * g + b


def _attn(xq, xkv, C, Wq, Wk, Wv, Wo):
    q = _mm(xq, Wq)
    k = _mm(xkv, Wk)
    v = _mm(xkv, Wv)
    outs = []
    inv = np.float32(1.0 / np.sqrt(DK))
    for h in range(H):
        qh = q[:, h * DK:(h + 1) * DK]
        kh = k[:, h * DK:(h + 1) * DK]
        vh = v[:, h * DK:(h + 1) * DK]
        S = lax.dot_general(qh.astype(jnp.bfloat16), kh.astype(jnp.bfloat16),
                            (((1,), (1,)), ((), ())),
                            preferred_element_type=jnp.float32) * inv
        W = jnp.exp(jnp.clip(S, -10.0, 10.0)) * C
        wv = _mm(W, vh)
        z = jnp.sum(W, axis=1, keepdims=True)
        outs.append(wv / (z + 1e-9))
    o = jnp.concatenate(outs, axis=1)
    return _mm(o, Wo)


def _ffn(x, W1, b1, W2, b2):
    h = jax.nn.relu(_mm(x, W1) + b1)
    return _mm(h, W2) + b2


def _body_kernel(enc_tree, dec_tree, n_enc, *refs):
    x_enc_ref, x_dec_ref, cnt_ref = refs[0], refs[1], refs[2]
    enc_refs = refs[3:3 + n_enc]
    dec_refs = refs[3 + n_enc:-1]
    out_ref = refs[-1]
    enc_params = jax.tree.unflatten(enc_tree, enc_refs)
    dec_params = jax.tree.unflatten(dec_tree, dec_refs)

    Cee = cnt_ref[0, 0 * N:1 * N] + cnt_ref[1, 0 * N:1 * N]
    Cdd = cnt_ref[0, 1 * N:2 * N] + cnt_ref[1, 1 * N:2 * N]
    Ced = cnt_ref[0, 2 * N:3 * N] + cnt_ref[1, 2 * N:3 * N]

    x = x_enc_ref[...]
    for p in enc_params:
        x = _layernorm(
            x + _attn(x, x, Cee, p['Wq'][...], p['Wk'][...], p['Wv'][...],
                      p['Wo'][...]),
            p['ln1_g'][...], p['ln1_b'][...])
        x = _layernorm(x + _ffn(x, p['W1'][...], p['b1'][...],
                                p['W2'][...], p['b2'][...]),
                       p['ln2_g'][...], p['ln2_b'][...])
    x_enc = x

    x = x_dec_ref[...]
    for p in dec_params:
        x = _layernorm(
            x + _attn(x, x, Cdd, p['Wq'][...], p['Wk'][...], p['Wv'][...],
                      p['Wo'][...]),
            p['ln1_g'][...], p['ln1_b'][...])
        x = _layernorm(
            x + _attn(x, x_enc, Ced, p['Wq2'][...], p['Wk2'][...],
                      p['Wv2'][...], p['Wo2'][...]),
            p['ln2_g'][...], p['ln2_b'][...])
        x = _layernorm(x + _ffn(x, p['W1'][...], p['b1'][...],
                                p['W2'][...], p['b2'][...]),
                       p['ln3_g'][...], p['ln3_b'][...])
    out_ref[...] = x.astype(jnp.bfloat16)


def _body(x_enc0, x_dec0, cnt, enc_params, dec_params):
    enc_leaves, enc_tree = jax.tree.flatten(enc_params)
    dec_leaves, dec_tree = jax.tree.flatten(dec_params)
    return pl.pallas_call(
        functools.partial(_body_kernel, enc_tree, dec_tree, len(enc_leaves)),
        out_shape=jax.ShapeDtypeStruct((N, D), jnp.bfloat16),
        compiler_params=pltpu.CompilerParams(
            vmem_limit_bytes=100 * 1024 * 1024),
    )(x_enc0, x_dec0, cnt, *enc_leaves, *dec_leaves)


BV = 3200
KV = VOCAB // BV


def _logz_kernel(x_ref, wg_ref, bg_ref, out_ref, m_sc, s_sc):
    j = pl.program_id(0)
    l = jnp.dot(x_ref[...], wg_ref[...],
                preferred_element_type=jnp.float32) + bg_ref[...]
    bm = jnp.max(l, axis=1, keepdims=True)

    @pl.when(j == 0)
    def _():
        m_sc[...] = bm
        s_sc[...] = jnp.sum(jnp.exp(l - bm), axis=1, keepdims=True)

    @pl.when(j > 0)
    def _():
        m_old = m_sc[...]
        m_new = jnp.maximum(m_old, bm)
        s_sc[...] = (s_sc[...] * jnp.exp(m_old - m_new)
                     + jnp.sum(jnp.exp(l - m_new), axis=1, keepdims=True))
        m_sc[...] = m_new

    @pl.when(j == KV - 1)
    def _():
        out_ref[...] = m_sc[...] + jnp.log(s_sc[...])


def _gen_out_kernel(x_ref, wg_ref, bg_ref, lz_ref, out_ref):
    l = jnp.dot(x_ref[...], wg_ref[...],
                preferred_element_type=jnp.float32) + bg_ref[...]
    out_ref[...] = l - lz_ref[...]


def _generator(xb, Wg, bg):
    wgb = Wg.astype(jnp.bfloat16)
    bg2 = bg.reshape(1, VOCAB)
    logz = pl.pallas_call(
        _logz_kernel,
        grid=(KV,),
        in_specs=[
            pl.BlockSpec((N, D), lambda j: (0, 0)),
            pl.BlockSpec((D, BV), lambda j: (0, j)),
            pl.BlockSpec((1, BV), lambda j: (0, j)),
        ],
        out_specs=pl.BlockSpec((N, 1), lambda j: (0, 0)),
        out_shape=jax.ShapeDtypeStruct((N, 1), jnp.float32),
        scratch_shapes=[pltpu.VMEM((N, 1), jnp.float32),
                        pltpu.VMEM((N, 1), jnp.float32)],
    )(xb, wgb, bg2)
    return pl.pallas_call(
        _gen_out_kernel,
        grid=(KV,),
        in_specs=[
            pl.BlockSpec((N, D), lambda j: (0, 0)),
            pl.BlockSpec((D, BV), lambda j: (0, j)),
            pl.BlockSpec((1, BV), lambda j: (0, j)),
            pl.BlockSpec((N, 1), lambda j: (0, 0)),
        ],
        out_specs=pl.BlockSpec((N, BV), lambda j: (0, j)),
        out_shape=jax.ShapeDtypeStruct((N, VOCAB), jnp.float32),
    )(xb, wgb, bg2, logz)


def kernel(params, src_tokens, src_pos, tgt_tokens, tgt_pos,
           ee_src, ee_dst, dd_src, dd_dst, ed_src, ed_dst):
    cnt_flat, x_enc0, x_dec0 = _sc_prep(
        ee_src, ee_dst, dd_src, dd_dst, ed_src, ed_dst,
        params['src_tok'], params['tgt_tok'], params['pos'],
        src_tokens, src_pos, tgt_tokens, tgt_pos)
    x_dec = _body(x_enc0, x_dec0, cnt_flat, params['enc'], params['dec'])
    return _generator(x_dec, params['Wg'], params['bg'])


# speedup vs baseline: 283.4006x; 1.0346x over previous
import functools

import jax
import jax.numpy as jnp
import numpy as np
from jax import lax
from jax.experimental import pallas as pl
from jax.experimental.pallas import tpu as pltpu
from jax.experimental.pallas import tpu_sc as plsc

H = 8
DK = 32
D = H * DK
VOCAB = 32000
DFF = 1024
N = 512
E = 65536

NC = 2
NS = 16
NW = NC * NS
EPW = E // NW
NBINS = N * N
TBINS = 3 * NBINS
SLICE = TBINS // NS
ROWS_PW = N // NW


def _histogram(c, s, wid, edge_lists, cnt_sh, srcbuf, dstbuf, idx_v,
               ones_v, sem):
    ebase = wid * EPW
    for t, (esrc, edst) in enumerate(edge_lists):
        pltpu.sync_copy(esrc.at[pl.ds(ebase, EPW)], srcbuf)
        pltpu.sync_copy(edst.at[pl.ds(ebase, EPW)], dstbuf)
        for j in range(16):
            for k in range(8):
                off = (j * 8 + k) * 16
                idx_v[j, pl.ds(k * 16, 16)] = (
                    dstbuf[pl.ds(off, 16)] * N
                    + srcbuf[pl.ds(off, 16)]
                    + t * NBINS)
        descs = [pltpu.async_copy(ones_v, cnt_sh.at[idx_v.at[j]], sem,
                                  add=True)
                 for j in range(16)]
        for d in descs:
            d.wait()


def _hist_out(c, s, ntypes, cnt_sh, cnt_out, sem):
    zrows = ntypes * N // NS
    odescs = [pltpu.async_copy(
        cnt_sh.at[pl.ds((s * zrows + r) * N, N)],
        cnt_out.at[c, s * zrows + r], sem)
        for r in range(zrows)]
    for dsc in odescs:
        dsc.wait()


def _sc1_body(ee_src, ee_dst,
              src_tok, tgt_tok, pos_tab,
              src_tokens, src_pos, tgt_tokens, tgt_pos,
              ones_in, zeros_in,
              cnt_out, x_enc_out, x_dec_out,
              srcbuf, dstbuf, idx_v, ones_v,
              tokidx, posidx, trows, prows, cnt_sh, sem):
    c = lax.axis_index("c")
    s = lax.axis_index("s")
    wid = c * NS + s

    pltpu.sync_copy(ones_in, ones_v)
    pltpu.sync_copy(zeros_in, cnt_sh.at[pl.ds(s * (NBINS // NS), NBINS // NS)])

    def _embed(tok_tab, tok_ids, pos_ids, out_ref):
        base = wid * ROWS_PW
        pltpu.sync_copy(tok_ids.at[pl.ds(base, ROWS_PW)], tokidx)
        pltpu.sync_copy(pos_ids.at[pl.ds(base, ROWS_PW)], posidx)
        pltpu.async_copy(tok_tab.at[tokidx], trows, sem).wait()
        pltpu.async_copy(pos_tab.at[posidx], prows, sem).wait()

        def _row(i, _):
            def _col(j, _):
                trows[i, pl.ds(j * 16, 16)] = (
                    trows[i, pl.ds(j * 16, 16)] + prows[i, pl.ds(j * 16, 16)])
                return 0
            lax.fori_loop(0, D // 16, _col, 0)
            return 0
        lax.fori_loop(0, ROWS_PW, _row, 0)
        pltpu.sync_copy(trows, out_ref.at[pl.ds(base, ROWS_PW)])

    _embed(src_tok, src_tokens, src_pos, x_enc_out)
    _embed(tgt_tok, tgt_tokens, tgt_pos, x_dec_out)

    plsc.subcore_barrier()
    _histogram(c, s, wid, ((ee_src, ee_dst),), cnt_sh,
               srcbuf, dstbuf, idx_v, ones_v, sem)
    plsc.subcore_barrier()
    _hist_out(c, s, 1, cnt_sh, cnt_out, sem)


def _sc2_body(dd_src, dd_dst, ed_src, ed_dst,
              ones_in, zeros_in,
              cnt_out,
              srcbuf, dstbuf, idx_v, ones_v, cnt_sh, sem):
    c = lax.axis_index("c")
    s = lax.axis_index("s")
    wid = c * NS + s

    pltpu.sync_copy(ones_in, ones_v)
    pltpu.sync_copy(zeros_in,
                    cnt_sh.at[pl.ds(s * (2 * NBINS // NS), 2 * NBINS // NS)])
    plsc.subcore_barrier()
    _histogram(c, s, wid, ((dd_src, dd_dst), (ed_src, ed_dst)), cnt_sh,
               srcbuf, dstbuf, idx_v, ones_v, sem)
    plsc.subcore_barrier()
    _hist_out(c, s, 2, cnt_sh, cnt_out, sem)


_EDGE_SCRATCH = (
    pltpu.VMEM((EPW,), jnp.int32),
    pltpu.VMEM((EPW,), jnp.int32),
    pltpu.VMEM((16, 128), jnp.int32),
    pltpu.VMEM((128,), jnp.float32),
)


def _sc_prep(ee_src, ee_dst, dd_src, dd_dst, ed_src, ed_dst,
             src_tok, tgt_tok, pos_tab,
             src_tokens, src_pos, tgt_tokens, tgt_pos):
    mesh = plsc.VectorSubcoreMesh(core_axis_name="c", subcore_axis_name="s",
                                  num_cores=NC, num_subcores=NS)
    ones_in = jnp.ones((128,), jnp.float32)

    f1 = pl.kernel(
        _sc1_body,
        out_type=(
            jax.ShapeDtypeStruct((NC, N, N), jnp.float32),
            jax.ShapeDtypeStruct((N, D), jnp.float32),
            jax.ShapeDtypeStruct((N, D), jnp.float32),
        ),
        mesh=mesh,
        scratch_types=_EDGE_SCRATCH + (
            pltpu.VMEM((ROWS_PW,), jnp.int32),
            pltpu.VMEM((ROWS_PW,), jnp.int32),
            pltpu.VMEM((ROWS_PW, D), jnp.float32),
            pltpu.VMEM((ROWS_PW, D), jnp.float32),
            pltpu.VMEM_SHARED((NBINS,), jnp.float32),
            pltpu.SemaphoreType.DMA,
        ),
    )
    cnt1, x_enc0, x_dec0 = f1(
        ee_src, ee_dst, src_tok, tgt_tok, pos_tab,
        src_tokens, src_pos, tgt_tokens, tgt_pos,
        ones_in, jnp.zeros((NBINS // NS,), jnp.float32))

    f2 = pl.kernel(
        _sc2_body,
        out_type=jax.ShapeDtypeStruct((NC, 2 * N, N), jnp.float32),
        mesh=mesh,
        scratch_types=_EDGE_SCRATCH + (
            pltpu.VMEM_SHARED((2 * NBINS,), jnp.float32),
            pltpu.SemaphoreType.DMA,
        ),
    )
    cnt2 = f2(dd_src, dd_dst, ed_src, ed_dst,
              ones_in, jnp.zeros((2 * NBINS // NS,), jnp.float32))
    return cnt1, cnt2, x_enc0, x_dec0


def _mm(a, b):
    return jnp.dot(a.astype(jnp.bfloat16), b.astype(jnp.bfloat16),
                   preferred_element_type=jnp.float32)


def _layernorm(x, g, b):
    m = jnp.mean(x, axis=-1, keepdims=True)
    v = jnp.mean((x - m) ** 2, axis=-1, keepdims=True)
    return (x - m) / jnp.sqrt(v + 1e-5) * g + b


def _attn(xq, xkv, C, Wq, Wk, Wv, Wo):
    inv = np.float32(1.0 / np.sqrt(DK))
    q = (_mm(xq, Wq) * inv).astype(jnp.bfloat16)
    k = _mm(xkv, Wk).astype(jnp.bfloat16)
    v = _mm(xkv, Wv)
    outs = []
    for h in range(H):
        qh = q[:, h * DK:(h + 1) * DK]
        kh = k[:, h * DK:(h + 1) * DK]
        vh = v[:, h * DK:(h + 1) * DK]
        S = lax.dot_general(qh, kh, (((1,), (1,)), ((), ())),
                            preferred_element_type=jnp.float32)
        W = jnp.exp(jnp.clip(S, -10.0, 10.0)) * C
        wv = _mm(W, vh)
        z = jnp.sum(W, axis=1, keepdims=True)
        outs.append(wv / (z + 1e-9))
    o = jnp.concatenate(outs, axis=1)
    return _mm(o, Wo)


def _ffn(x, W1, b1, W2, b2):
    h = jax.nn.relu(_mm(x, W1) + b1)
    return _mm(h, W2) + b2


def _enc_kernel(enc_tree, *refs):
    x_enc_ref, cnt_ref = refs[0], refs[1]
    out_ref = refs[-1]
    enc_params = jax.tree.unflatten(enc_tree, refs[2:-1])

    Cee = cnt_ref[0] + cnt_ref[1]
    x = x_enc_ref[...]
    for p in enc_params:
        x = _layernorm(
            x + _attn(x, x, Cee, p['Wq'][...], p['Wk'][...], p['Wv'][...],
                      p['Wo'][...]),
            p['ln1_g'][...], p['ln1_b'][...])
        x = _layernorm(x + _ffn(x, p['W1'][...], p['b1'][...],
                                p['W2'][...], p['b2'][...]),
                       p['ln2_g'][...], p['ln2_b'][...])
    out_ref[...] = x


def _dec_kernel(dec_tree, *refs):
    x_enc_ref, x_dec_ref, cnt_ref = refs[0], refs[1], refs[2]
    out_ref = refs[-1]
    dec_params = jax.tree.unflatten(dec_tree, refs[3:-1])

    Cdd = cnt_ref[0, 0 * N:1 * N] + cnt_ref[1, 0 * N:1 * N]
    Ced = cnt_ref[0, 1 * N:2 * N] + cnt_ref[1, 1 * N:2 * N]

    x_enc = x_enc_ref[...]
    x = x_dec_ref[...]
    for p in dec_params:
        x = _layernorm(
            x + _attn(x, x, Cdd, p['Wq'][...], p['Wk'][...], p['Wv'][...],
                      p['Wo'][...]),
            p['ln1_g'][...], p['ln1_b'][...])
        x = _layernorm(
            x + _attn(x, x_enc, Ced, p['Wq2'][...], p['Wk2'][...],
                      p['Wv2'][...], p['Wo2'][...]),
            p['ln2_g'][...], p['ln2_b'][...])
        x = _layernorm(x + _ffn(x, p['W1'][...], p['b1'][...],
                                p['W2'][...], p['b2'][...]),
                       p['ln3_g'][...], p['ln3_b'][...])
    out_ref[...] = x.astype(jnp.bfloat16)


_BODY_PARAMS = pltpu.CompilerParams(vmem_limit_bytes=100 * 1024 * 1024)


def _body(x_enc0, x_dec0, cnt1, cnt2, enc_params, dec_params):
    enc_leaves, enc_tree = jax.tree.flatten(enc_params)
    dec_leaves, dec_tree = jax.tree.flatten(dec_params)
    x_enc = pl.pallas_call(
        functools.partial(_enc_kernel, enc_tree),
        out_shape=jax.ShapeDtypeStruct((N, D), jnp.float32),
        compiler_params=_BODY_PARAMS,
    )(x_enc0, cnt1, *enc_leaves)
    return pl.pallas_call(
        functools.partial(_dec_kernel, dec_tree),
        out_shape=jax.ShapeDtypeStruct((N, D), jnp.bfloat16),
        compiler_params=_BODY_PARAMS,
    )(x_enc, x_dec0, cnt2, *dec_leaves)


BV = 3200
KV = VOCAB // BV


def _logz_kernel(x_ref, wg_ref, bg_ref, out_ref, m_sc, s_sc):
    j = pl.program_id(0)
    l = jnp.dot(x_ref[...], wg_ref[...],
                preferred_element_type=jnp.float32) + bg_ref[...]
    bm = jnp.max(l, axis=1, keepdims=True)

    @pl.when(j == 0)
    def _():
        m_sc[...] = bm
        s_sc[...] = jnp.sum(jnp.exp(l - bm), axis=1, keepdims=True)

    @pl.when(j > 0)
    def _():
        m_old = m_sc[...]
        m_new = jnp.maximum(m_old, bm)
        s_sc[...] = (s_sc[...] * jnp.exp(m_old - m_new)
                     + jnp.sum(jnp.exp(l - m_new), axis=1, keepdims=True))
        m_sc[...] = m_new

    @pl.when(j == KV - 1)
    def _():
        out_ref[...] = m_sc[...] + jnp.log(s_sc[...])


def _gen_out_kernel(x_ref, wg_ref, bg_ref, lz_ref, out_ref):
    l = jnp.dot(x_ref[...], wg_ref[...],
                preferred_element_type=jnp.float32) + bg_ref[...]
    out_ref[...] = l - lz_ref[...]


def _generator(xb, Wg, bg):
    wgb = Wg.astype(jnp.bfloat16)
    bg2 = bg.reshape(1, VOCAB)
    logz = pl.pallas_call(
        _logz_kernel,
        grid=(KV,),
        in_specs=[
            pl.BlockSpec((N, D), lambda j: (0, 0)),
            pl.BlockSpec((D, BV), lambda j: (0, j)),
            pl.BlockSpec((1, BV), lambda j: (0, j)),
        ],
        out_specs=pl.BlockSpec((N, 1), lambda j: (0, 0)),
        out_shape=jax.ShapeDtypeStruct((N, 1), jnp.float32),
        scratch_shapes=[pltpu.VMEM((N, 1), jnp.float32),
                        pltpu.VMEM((N, 1), jnp.float32)],
    )(xb, wgb, bg2)
    return pl.pallas_call(
        _gen_out_kernel,
        grid=(KV,),
        in_specs=[
            pl.BlockSpec((N, D), lambda j: (0, 0)),
            pl.BlockSpec((D, BV), lambda j: (0, j)),
            pl.BlockSpec((1, BV), lambda j: (0, j)),
            pl.BlockSpec((N, 1), lambda j: (0, 0)),
        ],
        out_specs=pl.BlockSpec((N, BV), lambda j: (0, j)),
        out_shape=jax.ShapeDtypeStruct((N, VOCAB), jnp.float32),
    )(xb, wgb, bg2, logz)


def kernel(params, src_tokens, src_pos, tgt_tokens, tgt_pos,
           ee_src, ee_dst, dd_src, dd_dst, ed_src, ed_dst):
    cnt1, cnt2, x_enc0, x_dec0 = _sc_prep(
        ee_src, ee_dst, dd_src, dd_dst, ed_src, ed_dst,
        params['src_tok'], params['tgt_tok'], params['pos'],
        src_tokens, src_pos, tgt_tokens, tgt_pos)
    x_dec = _body(x_enc0, x_dec0, cnt1, cnt2, params['enc'], params['dec'])
    return _generator(x_dec, params['Wg'], params['bg'])


# speedup vs baseline: 292.2677x; 1.0313x over previous
import functools

import jax
import jax.numpy as jnp
import numpy as np
from jax import lax
from jax.experimental import pallas as pl
from jax.experimental.pallas import tpu as pltpu
from jax.experimental.pallas import tpu_sc as plsc

H = 8
DK = 32
D = H * DK
VOCAB = 32000
DFF = 1024
N = 512
E = 65536

NC = 2
NS = 16
NW = NC * NS
EPW = E // NW
NBINS = N * N
TBINS = 3 * NBINS
SLICE = TBINS // NS
ROWS_PW = N // NW


def _histogram(c, s, wid, edge_lists, cnt_sh, srcbuf, dstbuf, idx_v,
               ones_v, sem):
    ebase = wid * EPW
    for t, (esrc, edst) in enumerate(edge_lists):
        pltpu.sync_copy(esrc.at[pl.ds(ebase, EPW)], srcbuf)
        pltpu.sync_copy(edst.at[pl.ds(ebase, EPW)], dstbuf)
        for j in range(16):
            for k in range(8):
                off = (j * 8 + k) * 16
                idx_v[j, pl.ds(k * 16, 16)] = (
                    dstbuf[pl.ds(off, 16)] * N
                    + srcbuf[pl.ds(off, 16)]
                    + t * NBINS)
        descs = [pltpu.async_copy(ones_v, cnt_sh.at[idx_v.at[j]], sem,
                                  add=True)
                 for j in range(16)]
        for d in descs:
            d.wait()


def _hist_out(c, s, ntypes, cnt_sh, cnt_out, sem):
    zrows = ntypes * N // NS
    odescs = [pltpu.async_copy(
        cnt_sh.at[pl.ds((s * zrows + r) * N, N)],
        cnt_out.at[c, s * zrows + r], sem)
        for r in range(zrows)]
    for dsc in odescs:
        dsc.wait()


def _sc1_body(ee_src, ee_dst,
              src_tok, tgt_tok, pos_tab,
              src_tokens, src_pos, tgt_tokens, tgt_pos,
              ones_in, zeros_in,
              cnt_out, xe_tok_out, xe_pos_out, xd_tok_out, xd_pos_out,
              srcbuf, dstbuf, idx_v, ones_v,
              tokidx, posidx, trows, prows, cnt_sh, sem, esem):
    c = lax.axis_index("c")
    s = lax.axis_index("s")
    wid = c * NS + s
    base = wid * ROWS_PW

    pltpu.sync_copy(ones_in, ones_v)
    pltpu.sync_copy(zeros_in, cnt_sh.at[pl.ds(s * (NBINS // NS), NBINS // NS)])
    plsc.subcore_barrier()

    ebase = wid * EPW
    pltpu.sync_copy(ee_src.at[pl.ds(ebase, EPW)], srcbuf)
    pltpu.sync_copy(ee_dst.at[pl.ds(ebase, EPW)], dstbuf)
    for j in range(16):
        for k in range(8):
            off = (j * 8 + k) * 16
            idx_v[j, pl.ds(k * 16, 16)] = (
                dstbuf[pl.ds(off, 16)] * N + srcbuf[pl.ds(off, 16)])
    descs = [pltpu.async_copy(ones_v, cnt_sh.at[idx_v.at[j]], sem, add=True)
             for j in range(16)]

    def _embed(tok_tab, tok_ids, pos_ids, tok_out, pos_out):
        pltpu.sync_copy(tok_ids.at[pl.ds(base, ROWS_PW)], tokidx)
        pltpu.sync_copy(pos_ids.at[pl.ds(base, ROWS_PW)], posidx)
        pltpu.async_copy(tok_tab.at[tokidx], trows, esem).wait()
        pltpu.async_copy(pos_tab.at[posidx], prows, esem).wait()
        pltpu.sync_copy(trows, tok_out.at[pl.ds(base, ROWS_PW)])
        pltpu.sync_copy(prows, pos_out.at[pl.ds(base, ROWS_PW)])

    _embed(src_tok, src_tokens, src_pos, xe_tok_out, xe_pos_out)
    _embed(tgt_tok, tgt_tokens, tgt_pos, xd_tok_out, xd_pos_out)

    for d in descs:
        d.wait()
    plsc.subcore_barrier()
    _hist_out(c, s, 1, cnt_sh, cnt_out, sem)


def _sc2_body(dd_src, dd_dst, ed_src, ed_dst,
              ones_in, zeros_in,
              cnt_out,
              srcbuf, dstbuf, idx_v, ones_v, cnt_sh, sem):
    c = lax.axis_index("c")
    s = lax.axis_index("s")
    wid = c * NS + s

    pltpu.sync_copy(ones_in, ones_v)
    pltpu.sync_copy(zeros_in,
                    cnt_sh.at[pl.ds(s * (2 * NBINS // NS), 2 * NBINS // NS)])
    plsc.subcore_barrier()
    _histogram(c, s, wid, ((dd_src, dd_dst), (ed_src, ed_dst)), cnt_sh,
               srcbuf, dstbuf, idx_v, ones_v, sem)
    plsc.subcore_barrier()
    _hist_out(c, s, 2, cnt_sh, cnt_out, sem)


_EDGE_SCRATCH = (
    pltpu.VMEM((EPW,), jnp.int32),
    pltpu.VMEM((EPW,), jnp.int32),
    pltpu.VMEM((16, 128), jnp.int32),
    pltpu.VMEM((128,), jnp.float32),
)


def _sc_prep(ee_src, ee_dst, dd_src, dd_dst, ed_src, ed_dst,
             src_tok, tgt_tok, pos_tab,
             src_tokens, src_pos, tgt_tokens, tgt_pos):
    mesh = plsc.VectorSubcoreMesh(core_axis_name="c", subcore_axis_name="s",
                                  num_cores=NC, num_subcores=NS)
    ones_in = jnp.ones((128,), jnp.float32)

    f1 = pl.kernel(
        _sc1_body,
        out_type=(
            jax.ShapeDtypeStruct((NC, N, N), jnp.float32),
            jax.ShapeDtypeStruct((N, D), jnp.float32),
            jax.ShapeDtypeStruct((N, D), jnp.float32),
            jax.ShapeDtypeStruct((N, D), jnp.float32),
            jax.ShapeDtypeStruct((N, D), jnp.float32),
        ),
        mesh=mesh,
        scratch_types=_EDGE_SCRATCH + (
            pltpu.VMEM((ROWS_PW,), jnp.int32),
            pltpu.VMEM((ROWS_PW,), jnp.int32),
            pltpu.VMEM((ROWS_PW, D), jnp.float32),
            pltpu.VMEM((ROWS_PW, D), jnp.float32),
            pltpu.VMEM_SHARED((NBINS,), jnp.float32),
            pltpu.SemaphoreType.DMA,
            pltpu.SemaphoreType.DMA,
        ),
    )
    cnt1, xe_tok, xe_pos, xd_tok, xd_pos = f1(
        ee_src, ee_dst, src_tok, tgt_tok, pos_tab,
        src_tokens, src_pos, tgt_tokens, tgt_pos,
        ones_in, jnp.zeros((NBINS // NS,), jnp.float32))

    f2 = pl.kernel(
        _sc2_body,
        out_type=jax.ShapeDtypeStruct((NC, 2 * N, N), jnp.float32),
        mesh=mesh,
        scratch_types=_EDGE_SCRATCH + (
            pltpu.VMEM_SHARED((2 * NBINS,), jnp.float32),
            pltpu.SemaphoreType.DMA,
        ),
    )
    cnt2 = f2(dd_src, dd_dst, ed_src, ed_dst,
              ones_in, jnp.zeros((2 * NBINS // NS,), jnp.float32))
    return cnt1, cnt2, xe_tok, xe_pos, xd_tok, xd_pos


def _mm(a, b):
    return jnp.dot(a.astype(jnp.bfloat16), b.astype(jnp.bfloat16),
                   preferred_element_type=jnp.float32)


def _layernorm(x, g, b):
    m = jnp.mean(x, axis=-1, keepdims=True)
    v = jnp.mean((x - m) ** 2, axis=-1, keepdims=True)
    return (x - m) / jnp.sqrt(v + 1e-5) * g + b


def _attn(xq, xkv, C, Wq, Wk, Wv, Wo):
    inv = np.float32(1.0 / np.sqrt(DK))
    q = (_mm(xq, Wq) * inv).astype(jnp.bfloat16)
    k = _mm(xkv, Wk).astype(jnp.bfloat16)
    v = _mm(xkv, Wv)
    outs = []
    for h in range(H):
        qh = q[:, h * DK:(h + 1) * DK]
        kh = k[:, h * DK:(h + 1) * DK]
        vh = v[:, h * DK:(h + 1) * DK]
        S = lax.dot_general(qh, kh, (((1,), (1,)), ((), ())),
                            preferred_element_type=jnp.float32)
        W = jnp.exp(jnp.clip(S, -10.0, 10.0)) * C
        wv = _mm(W, vh)
        z = jnp.sum(W, axis=1, keepdims=True)
        outs.append(wv / (z + 1e-9))
    o = jnp.concatenate(outs, axis=1)
    return _mm(o, Wo)


def _ffn(x, W1, b1, W2, b2):
    h = jax.nn.relu(_mm(x, W1) + b1)
    return _mm(h, W2) + b2


def _enc_kernel(enc_tree, *refs):
    xt_ref, xp_ref, cnt_ref = refs[0], refs[1], refs[2]
    out_ref = refs[-1]
    enc_params = jax.tree.unflatten(enc_tree, refs[3:-1])

    Cee = cnt_ref[0] + cnt_ref[1]
    x = xt_ref[...] + xp_ref[...]
    for p in enc_params:
        x = _layernorm(
            x + _attn(x, x, Cee, p['Wq'][...], p['Wk'][...], p['Wv'][...],
                      p['Wo'][...]),
            p['ln1_g'][...], p['ln1_b'][...])
        x = _layernorm(x + _ffn(x, p['W1'][...], p['b1'][...],
                                p['W2'][...], p['b2'][...]),
                       p['ln2_g'][...], p['ln2_b'][...])
    out_ref[...] = x


def _dec_kernel(dec_tree, *refs):
    x_enc_ref, xt_ref, xp_ref, cnt_ref = refs[0], refs[1], refs[2], refs[3]
    out_ref = refs[-1]
    dec_params = jax.tree.unflatten(dec_tree, refs[4:-1])

    Cdd = cnt_ref[0, 0 * N:1 * N] + cnt_ref[1, 0 * N:1 * N]
    Ced = cnt_ref[0, 1 * N:2 * N] + cnt_ref[1, 1 * N:2 * N]

    x_enc = x_enc_ref[...]
    x = xt_ref[...] + xp_ref[...]
    for p in dec_params:
        x = _layernorm(
            x + _attn(x, x, Cdd, p['Wq'][...], p['Wk'][...], p['Wv'][...],
                      p['Wo'][...]),
            p['ln1_g'][...], p['ln1_b'][...])
        x = _layernorm(
            x + _attn(x, x_enc, Ced, p['Wq2'][...], p['Wk2'][...],
                      p['Wv2'][...], p['Wo2'][...]),
            p['ln2_g'][...], p['ln2_b'][...])
        x = _layernorm(x + _ffn(x, p['W1'][...], p['b1'][...],
                                p['W2'][...], p['b2'][...]),
                       p['ln3_g'][...], p['ln3_b'][...])
    out_ref[...] = x.astype(jnp.bfloat16)


_BODY_PARAMS = pltpu.CompilerParams(vmem_limit_bytes=100 * 1024 * 1024)


def _body(xe_tok, xe_pos, xd_tok, xd_pos, cnt1, cnt2, enc_params, dec_params):
    enc_leaves, enc_tree = jax.tree.flatten(enc_params)
    dec_leaves, dec_tree = jax.tree.flatten(dec_params)
    x_enc = pl.pallas_call(
        functools.partial(_enc_kernel, enc_tree),
        out_shape=jax.ShapeDtypeStruct((N, D), jnp.float32),
        compiler_params=_BODY_PARAMS,
    )(xe_tok, xe_pos, cnt1, *enc_leaves)
    return pl.pallas_call(
        functools.partial(_dec_kernel, dec_tree),
        out_shape=jax.ShapeDtypeStruct((N, D), jnp.bfloat16),
        compiler_params=_BODY_PARAMS,
    )(x_enc, xd_tok, xd_pos, cnt2, *dec_leaves)


BV = 6400
KV = VOCAB // BV


def _logz_kernel(x_ref, wg_ref, bg_ref, out_ref, m_sc, s_sc):
    j = pl.program_id(0)
    l = jnp.dot(x_ref[...], wg_ref[...],
                preferred_element_type=jnp.float32) + bg_ref[...]
    bm = jnp.max(l, axis=1, keepdims=True)

    @pl.when(j == 0)
    def _():
        m_sc[...] = bm
        s_sc[...] = jnp.sum(jnp.exp(l - bm), axis=1, keepdims=True)

    @pl.when(j > 0)
    def _():
        m_old = m_sc[...]
        m_new = jnp.maximum(m_old, bm)
        s_sc[...] = (s_sc[...] * jnp.exp(m_old - m_new)
                     + jnp.sum(jnp.exp(l - m_new), axis=1, keepdims=True))
        m_sc[...] = m_new

    @pl.when(j == KV - 1)
    def _():
        out_ref[...] = m_sc[...] + jnp.log(s_sc[...])


def _gen_out_kernel(x_ref, wg_ref, bg_ref, lz_ref, out_ref):
    l = jnp.dot(x_ref[...], wg_ref[...],
                preferred_element_type=jnp.float32) + bg_ref[...]
    out_ref[...] = l - lz_ref[...]


def _generator(xb, Wg, bg):
    wgb = Wg.astype(jnp.bfloat16)
    bg2 = bg.reshape(1, VOCAB)
    logz = pl.pallas_call(
        _logz_kernel,
        grid=(KV,),
        in_specs=[
            pl.BlockSpec((N, D), lambda j: (0, 0)),
            pl.BlockSpec((D, BV), lambda j: (0, j)),
            pl.BlockSpec((1, BV), lambda j: (0, j)),
        ],
        out_specs=pl.BlockSpec((N, 1), lambda j: (0, 0)),
        out_shape=jax.ShapeDtypeStruct((N, 1), jnp.float32),
        scratch_shapes=[pltpu.VMEM((N, 1), jnp.float32),
                        pltpu.VMEM((N, 1), jnp.float32)],
    )(xb, wgb, bg2)
    return pl.pallas_call(
        _gen_out_kernel,
        grid=(KV,),
        in_specs=[
            pl.BlockSpec((N, D), lambda j: (0, 0)),
            pl.BlockSpec((D, BV), lambda j: (0, j)),
            pl.BlockSpec((1, BV), lambda j: (0, j)),
            pl.BlockSpec((N, 1), lambda j: (0, 0)),
        ],
        out_specs=pl.BlockSpec((N, BV), lambda j: (0, j)),
        out_shape=jax.ShapeDtypeStruct((N, VOCAB), jnp.float32),
    )(xb, wgb, bg2, logz)


def kernel(params, src_tokens, src_pos, tgt_tokens, tgt_pos,
           ee_src, ee_dst, dd_src, dd_dst, ed_src, ed_dst):
    cnt1, cnt2, xe_tok, xe_pos, xd_tok, xd_pos = _sc_prep(
        ee_src, ee_dst, dd_src, dd_dst, ed_src, ed_dst,
        params['src_tok'], params['tgt_tok'], params['pos'],
        src_tokens, src_pos, tgt_tokens, tgt_pos)
    x_dec = _body(xe_tok, xe_pos, xd_tok, xd_pos, cnt1, cnt2,
                  params['enc'], params['dec'])
    return _generator(x_dec, params['Wg'], params['bg'])
